# baseline (speedup 1.0000x reference)
import functools

import jax
import jax.numpy as jnp
from jax import lax
from jax.experimental import pallas as pl
from jax.experimental.pallas import tpu as pltpu
from jax.experimental.pallas import tpu_sc as plsc

B = 8192
D = 512
OUT_D = 3 * D
NW = 32
PW = B // NW
CHUNK = 16
NBUF = 2
LANES = 16


def _body(pos_hbm, d0_hbm, d1_hbm, d2_hbm, out_hbm, pos_v, obuf,
          d0_v, d1_v, d2_v, csem0, csem1, ssem0, ssem1):
    cid = lax.axis_index("c")
    sid = lax.axis_index("s")
    wid = sid * 2 + cid
    base = wid * PW

    @pl.when(sid == 0)
    def _stage_tables():
        pltpu.sync_copy(d0_hbm, d0_v)
        pltpu.sync_copy(d1_hbm, d1_v)
        pltpu.sync_copy(d2_hbm, d2_v)

    pltpu.sync_copy(pos_hbm.at[pl.ds(base, PW)], pos_v)
    plsc.subcore_barrier()

    ssems = (ssem0, ssem1)
    csems = (csem0, csem1)
    n_chunks = PW // CHUNK

    def chunk_body(c, _):
        b = c % NBUF
        dst = out_hbm.at[pl.ds(base + c * CHUNK, CHUNK)]
        pvec = pos_v[pl.ds(c * CHUNK, CHUNK)]
        tv = lax.shift_right_logical(pvec, 10)
        hv = jnp.bitwise_and(lax.shift_right_logical(pvec, 5), 31)
        wv = jnp.bitwise_and(pvec, 31)

        for bb in range(NBUF):
            @pl.when(b == bb)
            def _assemble(bb=bb):
                @pl.when(c >= NBUF)
                def _drain_store():
                    pltpu.make_async_copy(
                        obuf.at[pl.ds(bb * CHUNK, CHUNK)], dst, ssems[bb]
                    ).wait()

                for i in range(CHUNK):
                    row = bb * CHUNK + i
                    pltpu.async_copy(
                        d0_v.at[tv[i]], obuf.at[row, pl.ds(0, D)],
                        csems[bb])
                    pltpu.async_copy(
                        d1_v.at[hv[i]], obuf.at[row, pl.ds(D, D)],
                        csems[bb])
                    pltpu.async_copy(
                        d2_v.at[wv[i]], obuf.at[row, pl.ds(2 * D, D)],
                        csems[bb])

        @pl.when(c >= 1)
        def _store_prev():
            dst_prev = out_hbm.at[pl.ds(base + (c - 1) * CHUNK, CHUNK)]
            for bb in range(NBUF):
                @pl.when((1 - b) == bb)
                def _store(bb=bb):
                    src = obuf.at[pl.ds(bb * CHUNK, CHUNK)]
                    pltpu.make_async_copy(dst_prev, src, csems[bb]).wait()
                    pltpu.async_copy(src, dst_prev, ssems[bb])

        return 0

    lax.fori_loop(0, n_chunks, chunk_body, 0)

    last = n_chunks - 1
    bb_last = last % NBUF
    dst_last = out_hbm.at[pl.ds(base + last * CHUNK, CHUNK)]
    src_last = obuf.at[pl.ds(bb_last * CHUNK, CHUNK)]
    pltpu.make_async_copy(dst_last, src_last, csems[bb_last]).wait()
    pltpu.async_copy(src_last, dst_last, ssems[bb_last])
    pltpu.make_async_copy(
        obuf.at[pl.ds(0, CHUNK)], out_hbm.at[pl.ds(base, CHUNK)], ssem0
    ).wait()
    pltpu.make_async_copy(
        obuf.at[pl.ds(CHUNK, CHUNK)], out_hbm.at[pl.ds(base, CHUNK)], ssem1
    ).wait()


@jax.jit
def _run(position_ids, d_0, d_1, d_2):
    mesh = plsc.VectorSubcoreMesh(core_axis_name="c", subcore_axis_name="s")
    kern = functools.partial(
        pl.kernel,
        out_type=jax.ShapeDtypeStruct((B, OUT_D), jnp.float32),
        mesh=mesh,
        scratch_types=[
            pltpu.VMEM((PW,), jnp.int32),
            pltpu.VMEM((NBUF * CHUNK, OUT_D), jnp.float32),
            pltpu.VMEM_SHARED((16, D), jnp.float32),
            pltpu.VMEM_SHARED((32, D), jnp.float32),
            pltpu.VMEM_SHARED((32, D), jnp.float32),
            pltpu.SemaphoreType.DMA,
            pltpu.SemaphoreType.DMA,
            pltpu.SemaphoreType.DMA,
            pltpu.SemaphoreType.DMA,
        ],
    )(_body)
    return kern(position_ids.astype(jnp.int32), d_0, d_1, d_2)


def kernel(position_ids, d_0, d_1, d_2):
    out = _run(position_ids, d_0, d_1, d_2)
    return out[None]


# speedup vs baseline: 1.1820x; 1.1820x over previous
import functools

import jax
import jax.numpy as jnp
from jax import lax
from jax.experimental import pallas as pl
from jax.experimental.pallas import tpu as pltpu
from jax.experimental.pallas import tpu_sc as plsc

B = 8192
D = 512
OUT_D = 3 * D
NW = 32
PW = B // NW
CHUNK = 16
NBUF = 2
LANES = 16


def _body(pos_hbm, d0_hbm, d1_hbm, d2_hbm, out_hbm, pos_v, obuf,
          d0_v, d1_v, d2_v, csem, ssem0, ssem1):
    cid = lax.axis_index("c")
    sid = lax.axis_index("s")
    wid = sid * 2 + cid
    base = wid * PW

    @pl.when(sid == 0)
    def _stage_tables():
        pltpu.sync_copy(d0_hbm, d0_v)
        pltpu.sync_copy(d1_hbm, d1_v)
        pltpu.sync_copy(d2_hbm, d2_v)

    pltpu.sync_copy(pos_hbm.at[pl.ds(base, PW)], pos_v)
    plsc.subcore_barrier()

    ssems = (ssem0, ssem1)

    def chunk_body(c, _):
        b = c % NBUF
        row0 = b * CHUNK
        dst = out_hbm.at[pl.ds(base + c * CHUNK, CHUNK)]

        for bb in range(NBUF):
            @pl.when(jnp.logical_and(b == bb, c >= NBUF))
            def _drain_store(bb=bb):
                pltpu.make_async_copy(
                    obuf.at[pl.ds(bb * CHUNK, CHUNK)], dst, ssems[bb]
                ).wait()

        pvec = pos_v[pl.ds(c * CHUNK, CHUNK)]
        tv = lax.shift_right_logical(pvec, 10)
        hv = jnp.bitwise_and(lax.shift_right_logical(pvec, 5), 31)
        wv = jnp.bitwise_and(pvec, 31)
        for i in range(CHUNK):
            row = row0 + i
            pltpu.async_copy(
                d0_v.at[tv[i]], obuf.at[row, pl.ds(0, D)], csem)
            pltpu.async_copy(
                d1_v.at[hv[i]], obuf.at[row, pl.ds(D, D)], csem)
            pltpu.async_copy(
                d2_v.at[wv[i]], obuf.at[row, pl.ds(2 * D, D)], csem)

        pltpu.make_async_copy(
            dst, obuf.at[pl.ds(row0, CHUNK)], csem
        ).wait()

        for bb in range(NBUF):
            @pl.when(b == bb)
            def _store(bb=bb):
                pltpu.async_copy(
                    obuf.at[pl.ds(bb * CHUNK, CHUNK)], dst, ssems[bb])

        return 0

    lax.fori_loop(0, PW // CHUNK, chunk_body, 0)
    pltpu.make_async_copy(
        obuf.at[pl.ds(0, CHUNK)], out_hbm.at[pl.ds(base, CHUNK)], ssem0
    ).wait()
    pltpu.make_async_copy(
        obuf.at[pl.ds(CHUNK, CHUNK)], out_hbm.at[pl.ds(base, CHUNK)], ssem1
    ).wait()


@jax.jit
def _run(position_ids, d_0, d_1, d_2):
    mesh = plsc.VectorSubcoreMesh(core_axis_name="c", subcore_axis_name="s")
    kern = functools.partial(
        pl.kernel,
        out_type=jax.ShapeDtypeStruct((B, OUT_D), jnp.float32),
        mesh=mesh,
        scratch_types=[
            pltpu.VMEM((PW,), jnp.int32),
            pltpu.VMEM((NBUF * CHUNK, OUT_D), jnp.float32),
            pltpu.VMEM_SHARED((16, D), jnp.float32),
            pltpu.VMEM_SHARED((32, D), jnp.float32),
            pltpu.VMEM_SHARED((32, D), jnp.float32),
            pltpu.SemaphoreType.DMA,
            pltpu.SemaphoreType.DMA,
            pltpu.SemaphoreType.DMA,
        ],
    )(_body)
    return kern(position_ids.astype(jnp.int32), d_0, d_1, d_2)


def kernel(position_ids, d_0, d_1, d_2):
    out = _run(position_ids, d_0, d_1, d_2)
    return out[None]


# speedup vs baseline: 1.1884x; 1.0054x over previous
import functools

import jax
import jax.numpy as jnp
from jax import lax
from jax.experimental import pallas as pl
from jax.experimental.pallas import tpu as pltpu
from jax.experimental.pallas import tpu_sc as plsc

B = 8192
D = 512
OUT_D = 3 * D
NW = 32
PW = B // NW
CHUNK = 16
NBUF = 2
LANES = 16


def _body(pos_hbm, d0_hbm, d1_hbm, d2_hbm, out_hbm, pos_v, obuf,
          d0_v, d1_v, d2_v, csem, ssem0, ssem1):
    cid = lax.axis_index("c")
    sid = lax.axis_index("s")
    wid = sid * 2 + cid
    base = wid * PW

    @pl.when(sid == 0)
    def _stage_tables():
        pltpu.sync_copy(d0_hbm, d0_v)
        pltpu.sync_copy(d1_hbm, d1_v)
        pltpu.sync_copy(d2_hbm, d2_v)

    pltpu.sync_copy(pos_hbm.at[pl.ds(base, PW)], pos_v)
    plsc.subcore_barrier()

    ssems = (ssem0, ssem1)

    def chunk_body(c, _):
        b = c % NBUF
        row0 = b * CHUNK
        dst = out_hbm.at[pl.ds(base + c * CHUNK, CHUNK)]

        for bb in range(NBUF):
            @pl.when(jnp.logical_and(b == bb, c >= NBUF))
            def _drain_store(bb=bb):
                pltpu.make_async_copy(
                    obuf.at[pl.ds(bb * CHUNK, CHUNK)], dst, ssems[bb]
                ).wait()

        pvec = pos_v[pl.ds(c * CHUNK, CHUNK)]
        tv = lax.shift_right_logical(pvec, 10)
        hv = jnp.bitwise_and(lax.shift_right_logical(pvec, 5), 31)
        wv = jnp.bitwise_and(pvec, 31)
        for i in range(CHUNK):
            pltpu.async_copy(
                d0_v.at[tv[i]], obuf.at[row0 + i, pl.ds(0, D)], csem)
        for i in range(CHUNK):
            pltpu.async_copy(
                d1_v.at[hv[i]], obuf.at[row0 + i, pl.ds(D, D)], csem)
        for i in range(CHUNK):
            pltpu.async_copy(
                d2_v.at[wv[i]], obuf.at[row0 + i, pl.ds(2 * D, D)], csem)

        pltpu.make_async_copy(
            dst, obuf.at[pl.ds(row0, CHUNK)], csem
        ).wait()

        for bb in range(NBUF):
            @pl.when(b == bb)
            def _store(bb=bb):
                pltpu.async_copy(
                    obuf.at[pl.ds(bb * CHUNK, CHUNK)], dst, ssems[bb])

        return 0

    lax.fori_loop(0, PW // CHUNK, chunk_body, 0)
    pltpu.make_async_copy(
        obuf.at[pl.ds(0, CHUNK)], out_hbm.at[pl.ds(base, CHUNK)], ssem0
    ).wait()
    pltpu.make_async_copy(
        obuf.at[pl.ds(CHUNK, CHUNK)], out_hbm.at[pl.ds(base, CHUNK)], ssem1
    ).wait()


@jax.jit
def _run(position_ids, d_0, d_1, d_2):
    mesh = plsc.VectorSubcoreMesh(core_axis_name="c", subcore_axis_name="s")
    kern = functools.partial(
        pl.kernel,
        out_type=jax.ShapeDtypeStruct((B, OUT_D), jnp.float32),
        mesh=mesh,
        scratch_types=[
            pltpu.VMEM((PW,), jnp.int32),
            pltpu.VMEM((NBUF * CHUNK, OUT_D), jnp.float32),
            pltpu.VMEM_SHARED((16, D), jnp.float32),
            pltpu.VMEM_SHARED((32, D), jnp.float32),
            pltpu.VMEM_SHARED((32, D), jnp.float32),
            pltpu.SemaphoreType.DMA,
            pltpu.SemaphoreType.DMA,
            pltpu.SemaphoreType.DMA,
        ],
    )(_body)
    return kern(position_ids.astype(jnp.int32), d_0, d_1, d_2)


def kernel(position_ids, d_0, d_1, d_2):
    out = _run(position_ids, d_0, d_1, d_2)
    return out[None]
